# baseline (speedup 1.0000x reference)
import functools

import jax
import jax.numpy as jnp
from jax import lax
from jax.experimental import pallas as pl
from jax.experimental.pallas import tpu as pltpu
from jax.experimental.pallas import tpu_sc as plsc

VOCAB = 1000
DIM = 128
BATCH = 16384
FIELDS = 26
LN_EPS = 1e-5

NW = 32
BPW = BATCH // NW
CK = 128
NR = BPW // CK
NCH = FIELDS * NR
NBUF = 4


def _ln_table_kernel(w_ref, g_ref, b_ref, o_ref):
    w = w_ref[...]
    mean = jnp.mean(w, axis=1, keepdims=True)
    d = w - mean
    var = jnp.mean(d * d, axis=1, keepdims=True)
    o_ref[...] = d * lax.rsqrt(var + LN_EPS) * g_ref[...] + b_ref[...]


def _normalize_table(W, gamma, beta):
    return pl.pallas_call(
        _ln_table_kernel,
        out_shape=jax.ShapeDtypeStruct((VOCAB, DIM), jnp.float32),
    )(W, gamma.reshape(1, DIM), beta.reshape(1, DIM))


def _sc_gather_body(nt_hbm, xt_hbm, out_hbm, idx_t, buf_v,
                    g0, g1, g2, g3, s0, s1, s2, s3):
    nc = 2
    wid = lax.axis_index("s") * nc + lax.axis_index("c")
    row_base = wid * BPW
    gsem = (g0, g1, g2, g3)
    ssem = (s0, s1, s2, s3)

    pltpu.sync_copy(xt_hbm.at[pl.ds(0, FIELDS), pl.ds(row_base, BPW)], idx_t)

    def gather(c, b):
        f = c // NR
        r = c % NR
        return pltpu.make_async_copy(
            nt_hbm.at[idx_t.at[f, pl.ds(r * CK, CK)]], buf_v.at[b], gsem[b])

    def scatter(c, b):
        f = c // NR
        r = c % NR
        return pltpu.make_async_copy(
            buf_v.at[b],
            out_hbm.at[f, pl.ds(row_base + r * CK, CK)], ssem[b])

    def step(c, b, wait_s=True, start_g=True):
        gather(c, b).wait()
        if wait_s:
            scatter(c - 3, (b + 1) % NBUF).wait()
        if start_g:
            gather(c + 1, (b + 1) % NBUF).start()
        scatter(c, b).start()

    gather(0, 0).start()
    step(0, 0, wait_s=False)
    step(1, 1, wait_s=False)
    step(2, 2, wait_s=False)
    step(3, 3)

    def quad(g, _):
        c0 = 4 * g
        for k in range(NBUF):
            step(c0 + k, k)
        return 0

    lax.fori_loop(1, NCH // 4 - 1, quad, 0)

    step(NCH - 4, 0)
    step(NCH - 3, 1)
    step(NCH - 2, 2)
    step(NCH - 1, 3, start_g=False)
    scatter(NCH - 3, 1).wait()
    scatter(NCH - 2, 2).wait()
    scatter(NCH - 1, 3).wait()


@functools.partial(
    pl.kernel,
    out_type=jax.ShapeDtypeStruct((FIELDS, BATCH, DIM), jnp.float32),
    mesh=plsc.VectorSubcoreMesh(core_axis_name="c", subcore_axis_name="s"),
    scratch_types=[
        pltpu.VMEM((FIELDS, BPW), jnp.int32),
        pltpu.VMEM((NBUF, CK, DIM), jnp.float32),
        pltpu.SemaphoreType.DMA,
        pltpu.SemaphoreType.DMA,
        pltpu.SemaphoreType.DMA,
        pltpu.SemaphoreType.DMA,
        pltpu.SemaphoreType.DMA,
        pltpu.SemaphoreType.DMA,
        pltpu.SemaphoreType.DMA,
        pltpu.SemaphoreType.DMA,
    ],
)
def _sc_gather(nt_hbm, xt_hbm, out_hbm, idx_t, buf_v,
               g0, g1, g2, g3, s0, s1, s2, s3):
    _sc_gather_body(nt_hbm, xt_hbm, out_hbm, idx_t, buf_v,
                    g0, g1, g2, g3, s0, s1, s2, s3)


def kernel(x, W, gamma, beta):
    nt = _normalize_table(W, gamma, beta)
    out_fm = _sc_gather(nt, jnp.transpose(x.astype(jnp.int32)))
    return jnp.transpose(out_fm, (1, 0, 2))
